# Initial kernel scaffold; baseline (speedup 1.0000x reference)
import functools

import jax
import jax.numpy as jnp
from jax import lax
from jax.experimental import pallas as pl
from jax.experimental.pallas import tpu as pltpu
from jax.experimental.pallas import tpu_sc as plsc

F32 = jnp.float32


def _sc_pool(feats2, nidx_flat, valid, rel, *, B, N, Q, K, D, R):
  NSUB = 16
  G = 5
  QPS = Q // NSUB
  STEPS = QPS // G
  LJ = D // 16

  mesh = plsc.VectorSubcoreMesh(core_axis_name="c", subcore_axis_name="s")

  @functools.partial(
      pl.kernel,
      mesh=mesh,
      out_type=(
          jax.ShapeDtypeStruct((B, Q, D), F32),
          jax.ShapeDtypeStruct((B, Q, R), F32),
      ),
      scratch_types=[
          pltpu.VMEM((G * K,), jnp.int32),
          pltpu.VMEM((G * K, D), F32),
          pltpu.VMEM((G, K), jnp.int32),
          pltpu.VMEM((G, K, R), F32),
          pltpu.VMEM((G, D), F32),
          pltpu.VMEM((G, R), F32),
          pltpu.SemaphoreType.DMA,
      ],
  )
  def k(feats_hbm, nidx_hbm, valid_hbm, rel_hbm, sf_hbm, sr_hbm,
        idx_v, rows_v, valid_v, rel_v, sf_v, sr_v, sem):
    cid = lax.axis_index("c")
    sid = lax.axis_index("s")
    base_q = sid * QPS

    def step(i, _):
      qb = base_q + i * G
      pltpu.sync_copy(nidx_hbm.at[cid, pl.ds(qb * K, G * K)], idx_v)
      pltpu.sync_copy(valid_hbm.at[cid, pl.ds(qb, G)], valid_v)
      pltpu.sync_copy(rel_hbm.at[cid, pl.ds(qb, G)], rel_v)
      pltpu.async_copy(feats_hbm.at[idx_v], rows_v, sem).wait()

      for q in range(G):
        accs = [jnp.zeros((16,), F32) for _ in range(LJ)]
        racc = jnp.zeros((16,), F32)
        cnt = jnp.float32(0.0)
        for kk in range(K):
          s = valid_v[q, kk].astype(F32)
          cnt = cnt + s
          r = q * K + kk
          for j in range(LJ):
            accs[j] = accs[j] + rows_v[r, pl.ds(j * 16, 16)] * s
          racc = racc + rel_v[q, kk] * s
        inv = jnp.where(cnt == 0.0, 0.0, 1.0 / jnp.maximum(cnt, 1.0))
        for j in range(LJ):
          sf_v[q, pl.ds(j * 16, 16)] = accs[j] * inv
        sr_v[q] = racc * inv

      pltpu.sync_copy(sf_v, sf_hbm.at[cid, pl.ds(qb, G)])
      pltpu.sync_copy(sr_v, sr_hbm.at[cid, pl.ds(qb, G)])
      return 0

    lax.fori_loop(0, STEPS, step, 0)

  return k(feats2, nidx_flat, valid, rel)


def _tc_linear_relu(sf, sr, w1t, w2t, b2):
  M, D = sf.shape
  R = sr.shape[1]
  OUT = w1t.shape[1]
  BLK = 1000
  grid = (M // BLK,)

  def body(sf_ref, sr_ref, w1_ref, w2_ref, b_ref, o_ref):
    acc = jnp.dot(sf_ref[...], w1_ref[...], preferred_element_type=F32)
    acc = acc + jnp.dot(sr_ref[...], w2_ref[...], preferred_element_type=F32)
    o_ref[...] = jnp.maximum(acc + b_ref[...], 0.0)

  return pl.pallas_call(
      body,
      grid=grid,
      in_specs=[
          pl.BlockSpec((BLK, D), lambda i: (i, 0)),
          pl.BlockSpec((BLK, R), lambda i: (i, 0)),
          pl.BlockSpec((D, OUT), lambda i: (0, 0)),
          pl.BlockSpec((R, OUT), lambda i: (0, 0)),
          pl.BlockSpec((1, OUT), lambda i: (0, 0)),
      ],
      out_specs=pl.BlockSpec((BLK, OUT), lambda i: (i, 0)),
      out_shape=jax.ShapeDtypeStruct((M, OUT), F32),
  )(sf, sr, w1t, w2t, b2)


def kernel(keys, points, feats, n_idxs, neighbor_rel, neighbor_valid, W, b):
  B, N, D = feats.shape
  Q, K = n_idxs.shape[1], n_idxs.shape[2]
  R = neighbor_rel.shape[3]
  OUT = W.shape[0]

  feats2 = feats.reshape(B * N, D)
  nidx_flat = (n_idxs + (jnp.arange(B, dtype=jnp.int32) * N)[:, None, None]
               ).reshape(B, Q * K)

  sf, sr = _sc_pool(feats2, nidx_flat, neighbor_valid, neighbor_rel,
                    B=B, N=N, Q=Q, K=K, D=D, R=R)

  out = _tc_linear_relu(sf.reshape(B * Q, D), sr.reshape(B * Q, R),
                        W[:, :D].T, W[:, D:].T, b.reshape(1, OUT))
  return out.reshape(B, Q, OUT)


# speedup vs baseline: 9.2821x; 9.2821x over previous
import functools

import jax
import jax.numpy as jnp
from jax import lax
from jax.experimental import pallas as pl
from jax.experimental.pallas import tpu as pltpu
from jax.experimental.pallas import tpu_sc as plsc

F32 = jnp.float32


def _sc_pool(feats2, nidx_flat, valid, rel, *, B, N, Q, K, D, R):
  NSUB = 16
  G = 5
  QPS = Q // NSUB
  STEPS = QPS // G
  LJ = D // 16

  mesh = plsc.VectorSubcoreMesh(core_axis_name="c", subcore_axis_name="s")

  @functools.partial(
      pl.kernel,
      mesh=mesh,
      out_type=(
          jax.ShapeDtypeStruct((B * Q * D,), F32),
          jax.ShapeDtypeStruct((B * Q * R,), F32),
      ),
      scratch_types=[
          pltpu.VMEM((G * K,), jnp.int32),
          pltpu.VMEM((G * K, D), F32),
          pltpu.VMEM((G * K,), jnp.int32),
          pltpu.VMEM((G * K * R,), F32),
          pltpu.VMEM((G * D,), F32),
          pltpu.VMEM((G * R,), F32),
          pltpu.SemaphoreType.DMA,
      ],
  )
  def k(feats_hbm, nidx_hbm, valid_hbm, rel_hbm, sf_hbm, sr_hbm,
        idx_v, rows_v, valid_v, rel_v, sf_v, sr_v, sem):
    cid = lax.axis_index("c")
    sid = lax.axis_index("s")
    base_q = cid * Q + sid * QPS

    def step(i, _):
      qb = base_q + i * G
      ek = pl.multiple_of(qb * K, G * K)
      pltpu.sync_copy(nidx_hbm.at[pl.ds(ek, G * K)], idx_v)
      pltpu.sync_copy(valid_hbm.at[pl.ds(ek, G * K)], valid_v)
      pltpu.sync_copy(rel_hbm.at[pl.ds(pl.multiple_of(qb * K * R, G * K * R),
                                       G * K * R)], rel_v)
      pltpu.async_copy(feats_hbm.at[idx_v], rows_v, sem).wait()

      for q in range(G):
        accs = [jnp.zeros((16,), F32) for _ in range(LJ)]
        racc = jnp.zeros((16,), F32)
        vvq = valid_v[pl.ds(q * K, 16)].astype(F32)
        cnt = jnp.float32(0.0)
        for kk in range(K):
          s = vvq[kk]
          cnt = cnt + s
          r = q * K + kk
          for j in range(LJ):
            accs[j] = accs[j] + rows_v[r, pl.ds(j * 16, 16)] * s
          racc = racc + rel_v[pl.ds(r * R, 16)] * s
        cv = jnp.broadcast_to(cnt, (16,))
        inv = jnp.where(cv == 0.0, 0.0, 1.0 / jnp.maximum(cv, 1.0))
        for j in range(LJ):
          sf_v[pl.ds(q * D + j * 16, 16)] = accs[j] * inv
        sr_v[pl.ds(q * R, 16)] = racc * inv

      pltpu.sync_copy(sf_v, sf_hbm.at[pl.ds(pl.multiple_of(qb * D, G * D),
                                            G * D)])
      pltpu.sync_copy(sr_v, sr_hbm.at[pl.ds(pl.multiple_of(qb * R, G * R),
                                            G * R)])
      return 0

    lax.fori_loop(0, STEPS, step, 0)

  return k(feats2, nidx_flat, valid, rel)


def _tc_linear_relu(sf, sr, w1t, w2t, b2):
  M, D = sf.shape
  R = sr.shape[1]
  OUT = w1t.shape[1]
  BLK = 1000
  grid = (M // BLK,)

  def body(sf_ref, sr_ref, w1_ref, w2_ref, b_ref, o_ref):
    acc = jnp.dot(sf_ref[...], w1_ref[...], preferred_element_type=F32)
    acc = acc + jnp.dot(sr_ref[...], w2_ref[...], preferred_element_type=F32)
    o_ref[...] = jnp.maximum(acc + b_ref[...], 0.0)

  return pl.pallas_call(
      body,
      grid=grid,
      in_specs=[
          pl.BlockSpec((BLK, D), lambda i: (i, 0)),
          pl.BlockSpec((BLK, R), lambda i: (i, 0)),
          pl.BlockSpec((D, OUT), lambda i: (0, 0)),
          pl.BlockSpec((R, OUT), lambda i: (0, 0)),
          pl.BlockSpec((1, OUT), lambda i: (0, 0)),
      ],
      out_specs=pl.BlockSpec((BLK, OUT), lambda i: (i, 0)),
      out_shape=jax.ShapeDtypeStruct((M, OUT), F32),
  )(sf, sr, w1t, w2t, b2)


def kernel(keys, points, feats, n_idxs, neighbor_rel, neighbor_valid, W, b):
  B, N, D = feats.shape
  Q, K = n_idxs.shape[1], n_idxs.shape[2]
  R = neighbor_rel.shape[3]
  OUT = W.shape[0]

  feats2 = feats.reshape(B * N, D)
  nidx_flat = (n_idxs + (jnp.arange(B, dtype=jnp.int32) * N)[:, None, None]
               ).reshape(B * Q * K)

  sf, sr = _sc_pool(feats2, nidx_flat, neighbor_valid.reshape(B * Q * K),
                    neighbor_rel.reshape(B * Q * K * R),
                    B=B, N=N, Q=Q, K=K, D=D, R=R)

  out = _tc_linear_relu(sf.reshape(B * Q, D), sr.reshape(B * Q, R),
                        W[:, :D].T, W[:, D:].T, b.reshape(1, OUT))
  return out.reshape(B, Q, OUT)


# speedup vs baseline: 12.0175x; 1.2947x over previous
import functools

import jax
import jax.numpy as jnp
from jax import lax
from jax.experimental import pallas as pl
from jax.experimental.pallas import tpu as pltpu
from jax.experimental.pallas import tpu_sc as plsc

F32 = jnp.float32


def _sc_pool(feats2, nidx_flat, valid, rel, *, B, N, Q, K, D, R):
  NSUB = 16
  G = 5
  QPS = Q // NSUB
  STEPS = QPS // G
  LJ = D // 16

  mesh = plsc.VectorSubcoreMesh(core_axis_name="c", subcore_axis_name="s")

  @functools.partial(
      pl.kernel,
      mesh=mesh,
      out_type=(
          jax.ShapeDtypeStruct((B * Q * D,), F32),
          jax.ShapeDtypeStruct((B * Q * R,), F32),
      ),
      scratch_types=[
          pltpu.VMEM((G * K,), jnp.int32),
          pltpu.VMEM((G * K,), jnp.int32),
          pltpu.VMEM((G * K, D), F32),
          pltpu.VMEM((G * K, D), F32),
          pltpu.VMEM((G * K,), jnp.int32),
          pltpu.VMEM((G * K,), jnp.int32),
          pltpu.VMEM((G * K * R,), F32),
          pltpu.VMEM((G * K * R,), F32),
          pltpu.VMEM((G * D,), F32),
          pltpu.VMEM((G * D,), F32),
          pltpu.VMEM((G * R,), F32),
          pltpu.VMEM((G * R,), F32),
          pltpu.SemaphoreType.DMA,
          pltpu.SemaphoreType.DMA,
          pltpu.SemaphoreType.DMA,
          pltpu.SemaphoreType.DMA,
          pltpu.SemaphoreType.DMA,
          pltpu.SemaphoreType.DMA,
      ],
  )
  def k(feats_hbm, nidx_hbm, valid_hbm, rel_hbm, sf_hbm, sr_hbm,
        idx0, idx1, rows0, rows1, val0, val1, relv0, relv1,
        sf0, sf1, sr0, sr1, insem0, insem1, gsem0, gsem1, osem0, osem1):
    cid = lax.axis_index("c")
    sid = lax.axis_index("s")
    base_q = cid * Q + sid * QPS

    idx_v = [idx0, idx1]
    rows_v = [rows0, rows1]
    valid_v = [val0, val1]
    rel_v = [relv0, relv1]
    sf_v = [sf0, sf1]
    sr_v = [sr0, sr1]
    insem = [insem0, insem1]
    gsem = [gsem0, gsem1]
    osem = [osem0, osem1]

    def issue_in(step, b):
      qb = base_q + step * G
      ek = pl.multiple_of(qb * K, G * K)
      pltpu.async_copy(nidx_hbm.at[pl.ds(ek, G * K)], idx_v[b], insem[b])
      pltpu.async_copy(valid_hbm.at[pl.ds(ek, G * K)], valid_v[b], insem[b])
      pltpu.async_copy(
          rel_hbm.at[pl.ds(pl.multiple_of(qb * K * R, G * K * R), G * K * R)],
          rel_v[b], insem[b])

    def wait_in(b):
      pltpu.make_async_copy(nidx_hbm.at[pl.ds(0, G * K)], idx_v[b],
                            insem[b]).wait()
      pltpu.make_async_copy(valid_hbm.at[pl.ds(0, G * K)], valid_v[b],
                            insem[b]).wait()
      pltpu.make_async_copy(rel_hbm.at[pl.ds(0, G * K * R)], rel_v[b],
                            insem[b]).wait()

    def issue_gather(b):
      pltpu.async_copy(feats_hbm.at[idx_v[b]], rows_v[b], gsem[b])

    def wait_gather(b):
      pltpu.make_async_copy(feats_hbm.at[pl.ds(0, G * K)], rows_v[b],
                            gsem[b]).wait()

    def issue_out(step, b):
      qb = base_q + step * G
      pltpu.async_copy(
          sf_v[b],
          sf_hbm.at[pl.ds(pl.multiple_of(qb * D, G * D), G * D)], osem[b])
      pltpu.async_copy(
          sr_v[b],
          sr_hbm.at[pl.ds(pl.multiple_of(qb * R, G * R), G * R)], osem[b])

    def wait_out(b):
      pltpu.make_async_copy(sf_v[b], sf_hbm.at[pl.ds(0, G * D)],
                            osem[b]).wait()
      pltpu.make_async_copy(sr_v[b], sr_hbm.at[pl.ds(0, G * R)],
                            osem[b]).wait()

    def compute(b):
      for q in range(G):
        accs = [jnp.zeros((16,), F32) for _ in range(LJ)]
        racc = jnp.zeros((16,), F32)
        vvq = valid_v[b][pl.ds(q * K, 16)].astype(F32)
        cnt = jnp.float32(0.0)
        for kk in range(K):
          s = vvq[kk]
          cnt = cnt + s
          r = q * K + kk
          for j in range(LJ):
            accs[j] = accs[j] + rows_v[b][r, pl.ds(j * 16, 16)] * s
          racc = racc + rel_v[b][pl.ds(r * R, 16)] * s
        cv = jnp.broadcast_to(cnt, (16,))
        inv = jnp.where(cv == 0.0, 0.0, 1.0 / jnp.maximum(cv, 1.0))
        for j in range(LJ):
          sf_v[b][pl.ds(q * D + j * 16, 16)] = accs[j] * inv
        sr_v[b][pl.ds(q * R, 16)] = racc * inv

    issue_in(0, 0)
    wait_in(0)
    issue_gather(0)
    issue_in(1, 1)

    def pair(i, _):
      for b in (0, 1):
        s = i * 2 + b
        p, nq = b, 1 - b

        @pl.when(s < STEPS)
        def _():
          @pl.when(s + 1 < STEPS)
          def _():
            wait_in(nq)
            issue_gather(nq)

          wait_gather(p)

          @pl.when(s >= 2)
          def _():
            wait_out(p)

          compute(p)
          issue_out(s, p)

          @pl.when(s + 2 < STEPS)
          def _():
            issue_in(s + 2, p)
      return 0

    lax.fori_loop(0, (STEPS + 2) // 2, pair, 0)
    wait_out(0)
    wait_out(1)

  return k(feats2, nidx_flat, valid, rel)


def _tc_linear_relu(sf, sr, w1t, w2t, b2):
  M, D = sf.shape
  R = sr.shape[1]
  OUT = w1t.shape[1]
  BLK = 1000
  grid = (M // BLK,)

  def body(sf_ref, sr_ref, w1_ref, w2_ref, b_ref, o_ref):
    acc = jnp.dot(sf_ref[...], w1_ref[...], preferred_element_type=F32)
    acc = acc + jnp.dot(sr_ref[...], w2_ref[...], preferred_element_type=F32)
    o_ref[...] = jnp.maximum(acc + b_ref[...], 0.0)

  return pl.pallas_call(
      body,
      grid=grid,
      in_specs=[
          pl.BlockSpec((BLK, D), lambda i: (i, 0)),
          pl.BlockSpec((BLK, R), lambda i: (i, 0)),
          pl.BlockSpec((D, OUT), lambda i: (0, 0)),
          pl.BlockSpec((R, OUT), lambda i: (0, 0)),
          pl.BlockSpec((1, OUT), lambda i: (0, 0)),
      ],
      out_specs=pl.BlockSpec((BLK, OUT), lambda i: (i, 0)),
      out_shape=jax.ShapeDtypeStruct((M, OUT), F32),
  )(sf, sr, w1t, w2t, b2)


def kernel(keys, points, feats, n_idxs, neighbor_rel, neighbor_valid, W, b):
  B, N, D = feats.shape
  Q, K = n_idxs.shape[1], n_idxs.shape[2]
  R = neighbor_rel.shape[3]
  OUT = W.shape[0]

  feats2 = feats.reshape(B * N, D)
  nidx_flat = (n_idxs + (jnp.arange(B, dtype=jnp.int32) * N)[:, None, None]
               ).reshape(B * Q * K)

  sf, sr = _sc_pool(feats2, nidx_flat, neighbor_valid.reshape(B * Q * K),
                    neighbor_rel.reshape(B * Q * K * R),
                    B=B, N=N, Q=Q, K=K, D=D, R=R)

  out = _tc_linear_relu(sf.reshape(B * Q, D), sr.reshape(B * Q, R),
                        W[:, :D].T, W[:, D:].T, b.reshape(1, OUT))
  return out.reshape(B, Q, OUT)
